# Initial kernel scaffold; baseline (speedup 1.0000x reference)
import functools

import jax
import jax.numpy as jnp
from jax import lax
from jax.experimental import pallas as pl
from jax.experimental.pallas import tpu as pltpu
from jax.experimental.pallas import tpu_sc as plsc

N = 10000
E = 320000
D = 128
NC, NS, L = 2, 16, 16
NW = NC * NS
NP = 10240
EPW = E // NW
B = 80
NB = EPW // B
ZR = 128
RPS = NP // NS

_mesh = plsc.VectorSubcoreMesh(core_axis_name="c", subcore_axis_name="s")


@functools.partial(
    pl.kernel,
    out_type=jax.ShapeDtypeStruct((NC, NP), jnp.float32),
    mesh=_mesh,
    scratch_types=[
        pltpu.VMEM((B,), jnp.int32),
        pltpu.VMEM((B,), jnp.float32),
        pltpu.VMEM((RPS,), jnp.float32),
        pltpu.VMEM_SHARED((NP,), jnp.float32),
    ],
)
def _deg_kernel(col_hbm, ew_hbm, out_hbm, cidx, ewv, zsrc, acc):
    cid = lax.axis_index("c")
    sid = lax.axis_index("s")

    def zbody(i, carry):
        zsrc[pl.ds(i * L, L)] = jnp.zeros((L,), jnp.float32)
        return carry

    lax.fori_loop(0, RPS // L, zbody, 0)
    pltpu.sync_copy(zsrc, acc.at[pl.ds(sid * RPS, RPS)])
    plsc.subcore_barrier()

    wid = cid * NS + sid

    def body(i, carry):
        base = wid * EPW + i * B
        pltpu.sync_copy(col_hbm.at[pl.ds(base, B)], cidx)
        pltpu.sync_copy(ew_hbm.at[pl.ds(base, B)], ewv)
        pltpu.sync_copy(ewv, acc.at[cidx], add=True)
        return carry

    lax.fori_loop(0, NB, body, 0)
    plsc.subcore_barrier()
    pltpu.sync_copy(acc.at[pl.ds(sid * RPS, RPS)],
                    out_hbm.at[cid, pl.ds(sid * RPS, RPS)])


@functools.partial(
    pl.kernel,
    out_type=jax.ShapeDtypeStruct((NC, NP, D), jnp.float32),
    mesh=_mesh,
    scratch_types=[
        pltpu.VMEM((B,), jnp.int32),
        pltpu.VMEM((B,), jnp.int32),
        pltpu.VMEM((B,), jnp.float32),
        pltpu.VMEM((B, D), jnp.float32),
        pltpu.VMEM((ZR, D), jnp.float32),
        pltpu.VMEM_SHARED((NP, D), jnp.float32),
        pltpu.SemaphoreType.DMA,
    ],
)
def _edge_agg_kernel(xs_hbm, row_hbm, col_hbm, ew_hbm, out_hbm,
                     ridx, cidx, ewv, rows, zsrc, acc, gsem):
    cid = lax.axis_index("c")
    sid = lax.axis_index("s")

    def zbody(i, carry):
        for k in range(D // L):
            zsrc[i, pl.ds(k * L, L)] = jnp.zeros((L,), jnp.float32)
        return carry

    lax.fori_loop(0, ZR, zbody, 0)
    for j in range(RPS // ZR):
        pltpu.sync_copy(zsrc, acc.at[pl.ds(sid * RPS + j * ZR, ZR)])
    plsc.subcore_barrier()

    wid = cid * NS + sid

    def body(i, carry):
        base = wid * EPW + i * B
        pltpu.sync_copy(row_hbm.at[pl.ds(base, B)], ridx)
        pltpu.sync_copy(col_hbm.at[pl.ds(base, B)], cidx)
        pltpu.sync_copy(ew_hbm.at[pl.ds(base, B)], ewv)
        pltpu.async_copy(xs_hbm.at[ridx], rows, gsem).wait()

        def scale(e, c2):
            w = ewv[e]
            for k in range(D // L):
                sl = pl.ds(k * L, L)
                rows[e, sl] = rows[e, sl] * w
            return c2

        lax.fori_loop(0, B, scale, 0)
        pltpu.sync_copy(rows, acc.at[cidx], add=True)
        return carry

    lax.fori_loop(0, NB, body, 0)
    plsc.subcore_barrier()
    pltpu.sync_copy(acc.at[pl.ds(sid * RPS, RPS)],
                    out_hbm.at[cid, pl.ds(sid * RPS, RPS)])


def _b0_body(degp_ref, x_ref, w_ref, dinv_ref, xs_ref):
    deg = degp_ref[0, :N] + degp_ref[1, :N] + 1.0
    dinv = jnp.where(deg > 0, lax.rsqrt(deg), 0.0)[:, None]
    dinv_ref[...] = dinv
    xw = jnp.dot(x_ref[...], w_ref[...], preferred_element_type=jnp.float32)
    xs_ref[...] = dinv * xw


_b0 = pl.pallas_call(
    _b0_body,
    out_shape=(
        jax.ShapeDtypeStruct((N, 1), jnp.float32),
        jax.ShapeDtypeStruct((N, D), jnp.float32),
    ),
)


def _bmid_body(aggp_ref, xs_ref, dinv_ref, b_ref, w_ref, xsn_ref):
    dinv = dinv_ref[...]
    agg = aggp_ref[0, :N, :] + aggp_ref[1, :N, :] + xs_ref[...]
    h = jnp.maximum(dinv * agg + b_ref[...][None, :], 0.0)
    xw = jnp.dot(h, w_ref[...], preferred_element_type=jnp.float32)
    xsn_ref[...] = dinv * xw


_bmid = pl.pallas_call(
    _bmid_body,
    out_shape=jax.ShapeDtypeStruct((N, D), jnp.float32),
)


def _bfin_body(aggp_ref, xs_ref, dinv_ref, b_ref, out_ref):
    agg = aggp_ref[0, :N, :] + aggp_ref[1, :N, :] + xs_ref[...]
    out_ref[...] = dinv_ref[...] * agg + b_ref[...][None, :]


_bfin = pl.pallas_call(
    _bfin_body,
    out_shape=jax.ShapeDtypeStruct((N, D), jnp.float32),
)


def kernel(x, edge_index, edge_weight, W1, b1, W2, b2, W3, b3):
    row = edge_index[0].astype(jnp.int32)
    col = edge_index[1].astype(jnp.int32)
    ew = edge_weight.astype(jnp.float32)

    degp = _deg_kernel(col, ew)
    dinv, xs = _b0(degp, x, W1)
    aggp = _edge_agg_kernel(xs, row, col, ew)
    xs = _bmid(aggp, xs, dinv, b1, W2)
    aggp = _edge_agg_kernel(xs, row, col, ew)
    xs = _bmid(aggp, xs, dinv, b2, W3)
    aggp = _edge_agg_kernel(xs, row, col, ew)
    return _bfin(aggp, xs, dinv, b3)


# speedup vs baseline: 8.0245x; 8.0245x over previous
import functools

import jax
import jax.numpy as jnp
from jax import lax
from jax.experimental import pallas as pl
from jax.experimental.pallas import tpu as pltpu
from jax.experimental.pallas import tpu_sc as plsc

N = 10000
E = 320000
D = 128
NC, NS, L = 2, 16, 16
NW = NC * NS
NP = 10240
EPW = E // NW
B = 80
NB = EPW // B
ZR = 128
RPS = NP // NS

_mesh = plsc.VectorSubcoreMesh(core_axis_name="c", subcore_axis_name="s")


@functools.partial(
    pl.kernel,
    out_type=jax.ShapeDtypeStruct((NC, NP), jnp.float32),
    mesh=_mesh,
    scratch_types=[
        pltpu.VMEM((B,), jnp.int32),
        pltpu.VMEM((B,), jnp.float32),
        pltpu.VMEM((RPS,), jnp.float32),
        pltpu.VMEM_SHARED((NP,), jnp.float32),
    ],
)
def _deg_kernel(col_hbm, ew_hbm, out_hbm, cidx, ewv, zsrc, acc):
    cid = lax.axis_index("c")
    sid = lax.axis_index("s")

    def zbody(i, carry):
        zsrc[pl.ds(i * L, L)] = jnp.zeros((L,), jnp.float32)
        return carry

    lax.fori_loop(0, RPS // L, zbody, 0)
    pltpu.sync_copy(zsrc, acc.at[pl.ds(sid * RPS, RPS)])
    plsc.subcore_barrier()

    wid = cid * NS + sid

    def body(i, carry):
        base = wid * EPW + i * B
        pltpu.sync_copy(col_hbm.at[pl.ds(base, B)], cidx)
        pltpu.sync_copy(ew_hbm.at[pl.ds(base, B)], ewv)
        pltpu.sync_copy(ewv, acc.at[cidx], add=True)
        return carry

    lax.fori_loop(0, NB, body, 0)
    plsc.subcore_barrier()
    pltpu.sync_copy(acc.at[pl.ds(sid * RPS, RPS)],
                    out_hbm.at[cid, pl.ds(sid * RPS, RPS)])


@functools.partial(
    pl.kernel,
    out_type=jax.ShapeDtypeStruct((NC, NP, D), jnp.float32),
    mesh=_mesh,
    scratch_types=[
        pltpu.VMEM((B,), jnp.int32),
        pltpu.VMEM((B,), jnp.int32),
        pltpu.VMEM((B + L,), jnp.float32),
        pltpu.VMEM((B, D), jnp.float32),
        pltpu.VMEM((ZR, D), jnp.float32),
        pltpu.VMEM_SHARED((NP, D), jnp.float32),
        pltpu.SemaphoreType.DMA,
    ],
)
def _edge_agg_kernel(xs_hbm, row_hbm, col_hbm, ew_hbm, out_hbm,
                     ridx, cidx, ewv, rows, zsrc, acc, gsem):
    cid = lax.axis_index("c")
    sid = lax.axis_index("s")

    def zbody(i, carry):
        for k in range(D // L):
            zsrc[i, pl.ds(k * L, L)] = jnp.zeros((L,), jnp.float32)
        return carry

    lax.fori_loop(0, ZR, zbody, 0)
    for j in range(RPS // ZR):
        pltpu.sync_copy(zsrc, acc.at[pl.ds(sid * RPS + j * ZR, ZR)])
    plsc.subcore_barrier()

    wid = cid * NS + sid

    def body(i, carry):
        base = wid * EPW + i * B
        pltpu.sync_copy(row_hbm.at[pl.ds(base, B)], ridx)
        pltpu.sync_copy(col_hbm.at[pl.ds(base, B)], cidx)
        pltpu.sync_copy(ew_hbm.at[pl.ds(base, B)], ewv.at[pl.ds(0, B)])
        pltpu.async_copy(xs_hbm.at[ridx], rows, gsem).wait()

        def scale(e, c2):
            w = ewv[pl.ds(e, L)][0]
            for k in range(D // L):
                sl = pl.ds(k * L, L)
                rows[e, sl] = rows[e, sl] * w
            return c2

        lax.fori_loop(0, B, scale, 0)
        pltpu.sync_copy(rows, acc.at[cidx], add=True)
        return carry

    lax.fori_loop(0, NB, body, 0)
    plsc.subcore_barrier()
    pltpu.sync_copy(acc.at[pl.ds(sid * RPS, RPS)],
                    out_hbm.at[cid, pl.ds(sid * RPS, RPS)])


def _b0_body(degp_ref, x_ref, w_ref, dinv_ref, xs_ref):
    deg = degp_ref[0, :N] + degp_ref[1, :N] + 1.0
    dinv = jnp.where(deg > 0, lax.rsqrt(deg), 0.0)[:, None]
    dinv_ref[...] = dinv
    xw = jnp.dot(x_ref[...], w_ref[...], preferred_element_type=jnp.float32)
    xs_ref[...] = dinv * xw


_b0 = pl.pallas_call(
    _b0_body,
    out_shape=(
        jax.ShapeDtypeStruct((N, 1), jnp.float32),
        jax.ShapeDtypeStruct((N, D), jnp.float32),
    ),
)


def _bmid_body(aggp_ref, xs_ref, dinv_ref, b_ref, w_ref, xsn_ref):
    dinv = dinv_ref[...]
    agg = aggp_ref[0, :N, :] + aggp_ref[1, :N, :] + xs_ref[...]
    h = jnp.maximum(dinv * agg + b_ref[...][None, :], 0.0)
    xw = jnp.dot(h, w_ref[...], preferred_element_type=jnp.float32)
    xsn_ref[...] = dinv * xw


_bmid = pl.pallas_call(
    _bmid_body,
    out_shape=jax.ShapeDtypeStruct((N, D), jnp.float32),
)


def _bfin_body(aggp_ref, xs_ref, dinv_ref, b_ref, out_ref):
    agg = aggp_ref[0, :N, :] + aggp_ref[1, :N, :] + xs_ref[...]
    out_ref[...] = dinv_ref[...] * agg + b_ref[...][None, :]


_bfin = pl.pallas_call(
    _bfin_body,
    out_shape=jax.ShapeDtypeStruct((N, D), jnp.float32),
)


def kernel(x, edge_index, edge_weight, W1, b1, W2, b2, W3, b3):
    row = edge_index[0].astype(jnp.int32)
    col = edge_index[1].astype(jnp.int32)
    ew = edge_weight.astype(jnp.float32)

    degp = _deg_kernel(col, ew)
    dinv, xs = _b0(degp, x, W1)
    aggp = _edge_agg_kernel(xs, row, col, ew)
    xs = _bmid(aggp, xs, dinv, b1, W2)
    aggp = _edge_agg_kernel(xs, row, col, ew)
    xs = _bmid(aggp, xs, dinv, b2, W3)
    aggp = _edge_agg_kernel(xs, row, col, ew)
    return _bfin(aggp, xs, dinv, b3)


# speedup vs baseline: 8.8322x; 1.1007x over previous
import functools

import jax
import jax.numpy as jnp
from jax import lax
from jax.experimental import pallas as pl
from jax.experimental.pallas import tpu as pltpu
from jax.experimental.pallas import tpu_sc as plsc

N = 10000
E = 320000
D = 128
NC, NS, L = 2, 16, 16
NW = NC * NS
NP = 10240
EPW = E // NW
B = 128
NB = 80
EPP = NB * B
ZR = 128
RPS = NP // NS

_mesh = plsc.VectorSubcoreMesh(core_axis_name="c", subcore_axis_name="s")


@functools.partial(
    pl.kernel,
    out_type=jax.ShapeDtypeStruct((NC, NP), jnp.float32),
    mesh=_mesh,
    scratch_types=[
        pltpu.VMEM((NB, B), jnp.int32),
        pltpu.VMEM((NB, B), jnp.float32),
        pltpu.VMEM((RPS,), jnp.float32),
        pltpu.VMEM_SHARED((NP,), jnp.float32),
    ],
)
def _deg_kernel(col_hbm, ew_hbm, out_hbm, cidx, ewv, zsrc, acc):
    cid = lax.axis_index("c")
    sid = lax.axis_index("s")

    def zbody(i, carry):
        zsrc[pl.ds(i * L, L)] = jnp.zeros((L,), jnp.float32)
        return carry

    lax.fori_loop(0, RPS // L, zbody, 0)
    pltpu.sync_copy(zsrc, acc.at[pl.ds(sid * RPS, RPS)])

    wid = cid * NS + sid
    pltpu.sync_copy(col_hbm.at[wid], cidx)
    pltpu.sync_copy(ew_hbm.at[wid], ewv)
    plsc.subcore_barrier()

    def body(i, carry):
        pltpu.sync_copy(ewv.at[i], acc.at[cidx.at[i]], add=True)
        return carry

    lax.fori_loop(0, NB, body, 0)
    plsc.subcore_barrier()
    pltpu.sync_copy(acc.at[pl.ds(sid * RPS, RPS)],
                    out_hbm.at[cid, pl.ds(sid * RPS, RPS)])


@functools.partial(
    pl.kernel,
    out_type=jax.ShapeDtypeStruct((NC, NP, D), jnp.float32),
    mesh=_mesh,
    scratch_types=[
        pltpu.VMEM((2, B), jnp.int32),
        pltpu.VMEM((2, B), jnp.int32),
        pltpu.VMEM((2, B), jnp.float32),
        pltpu.VMEM((B, D), jnp.float32),
        pltpu.VMEM((B, D), jnp.float32),
        pltpu.VMEM_SHARED((NP, D), jnp.float32),
        pltpu.SemaphoreType.DMA,
        pltpu.SemaphoreType.DMA,
        pltpu.SemaphoreType.DMA,
        pltpu.SemaphoreType.DMA,
    ],
)
def _edge_agg_kernel(xs_hbm, row_hbm, col_hbm, ew_hbm, out_hbm,
                     ridx, cidx, ewv, gb0, gb1, acc,
                     gsem0, gsem1, isem0, isem1):
    cid = lax.axis_index("c")
    sid = lax.axis_index("s")
    gbufs = (gb0, gb1)
    gsems = (gsem0, gsem1)
    isems = (isem0, isem1)

    def zbody(i, carry):
        for k in range(D // L):
            gb0[i, pl.ds(k * L, L)] = jnp.zeros((L,), jnp.float32)
        return carry

    lax.fori_loop(0, ZR, zbody, 0)
    for j in range(RPS // ZR):
        pltpu.sync_copy(gb0, acc.at[pl.ds(sid * RPS + j * ZR, ZR)])
    plsc.subcore_barrier()

    wid = cid * NS + sid

    def fire_idx(i, s, sem):
        pltpu.async_copy(row_hbm.at[wid, i], ridx.at[s], sem)
        pltpu.async_copy(col_hbm.at[wid, i], cidx.at[s], sem)
        pltpu.async_copy(ew_hbm.at[wid, i], ewv.at[s], sem)

    def wait_idx(s, sem):
        pltpu.make_async_copy(row_hbm.at[wid, 0], ridx.at[s], sem).wait()
        pltpu.make_async_copy(col_hbm.at[wid, 0], cidx.at[s], sem).wait()
        pltpu.make_async_copy(ew_hbm.at[wid, 0], ewv.at[s], sem).wait()

    fire_idx(0, 0, isem0)
    fire_idx(1, 1, isem1)
    wait_idx(0, isem0)
    pltpu.async_copy(xs_hbm.at[ridx.at[0]], gb0, gsem0)

    def body(i2, carry):
        for b in range(2):
            i = i2 * 2 + b
            gb, gsem = gbufs[b], gsems[b]
            pltpu.make_async_copy(xs_hbm.at[ridx.at[b]], gb, gsem).wait()

            @pl.when(i + 1 < NB)
            def _():
                wait_idx(1 - b, isems[1 - b])
                pltpu.async_copy(xs_hbm.at[ridx.at[1 - b]],
                                 gbufs[1 - b], gsems[1 - b])

            def scale(g, c2):
                w_grp = ewv[b, pl.ds(g * L, L)]
                for j in range(L):
                    w = w_grp[j]
                    e = g * L + j
                    for k in range(D // L):
                        sl = pl.ds(k * L, L)
                        gb[e, sl] = gb[e, sl] * w
                return c2

            lax.fori_loop(0, B // L, scale, 0)

            pltpu.sync_copy(gb, acc.at[cidx.at[b]], add=True)

            @pl.when(i + 2 < NB)
            def _():
                fire_idx(i + 2, b, isems[b])
        return carry

    lax.fori_loop(0, NB // 2, body, 0)

    plsc.subcore_barrier()
    pltpu.sync_copy(acc.at[pl.ds(sid * RPS, RPS)],
                    out_hbm.at[cid, pl.ds(sid * RPS, RPS)])


def _b0_body(degp_ref, x_ref, w_ref, dinv_ref, xs_ref):
    deg = degp_ref[0, :N] + degp_ref[1, :N] + 1.0
    dinv = jnp.where(deg > 0, lax.rsqrt(deg), 0.0)[:, None]
    dinv_ref[...] = dinv
    xw = jnp.dot(x_ref[...], w_ref[...], preferred_element_type=jnp.float32)
    xs_ref[...] = dinv * xw


_b0 = pl.pallas_call(
    _b0_body,
    out_shape=(
        jax.ShapeDtypeStruct((N, 1), jnp.float32),
        jax.ShapeDtypeStruct((N, D), jnp.float32),
    ),
)


def _bmid_body(aggp_ref, xs_ref, dinv_ref, b_ref, w_ref, xsn_ref):
    dinv = dinv_ref[...]
    agg = aggp_ref[0, :N, :] + aggp_ref[1, :N, :] + xs_ref[...]
    h = jnp.maximum(dinv * agg + b_ref[...][None, :], 0.0)
    xw = jnp.dot(h, w_ref[...], preferred_element_type=jnp.float32)
    xsn_ref[...] = dinv * xw


_bmid = pl.pallas_call(
    _bmid_body,
    out_shape=jax.ShapeDtypeStruct((N, D), jnp.float32),
)


def _bfin_body(aggp_ref, xs_ref, dinv_ref, b_ref, out_ref):
    agg = aggp_ref[0, :N, :] + aggp_ref[1, :N, :] + xs_ref[...]
    out_ref[...] = dinv_ref[...] * agg + b_ref[...][None, :]


_bfin = pl.pallas_call(
    _bfin_body,
    out_shape=jax.ShapeDtypeStruct((N, D), jnp.float32),
)


def kernel(x, edge_index, edge_weight, W1, b1, W2, b2, W3, b3):
    row = edge_index[0].astype(jnp.int32)
    col = edge_index[1].astype(jnp.int32)
    ew = edge_weight.astype(jnp.float32)

    pad = EPP - EPW

    def chunked(a, fill):
        a2 = a.reshape(NW, EPW)
        a2 = jnp.pad(a2, ((0, 0), (0, pad)), constant_values=fill)
        return a2.reshape(NW, NB, B)

    row3 = chunked(row, 0)
    col3 = chunked(col, NP - 1)
    ew3 = chunked(ew, 0.0)

    degp = _deg_kernel(col3, ew3)
    dinv, xs = _b0(degp, x, W1)
    aggp = _edge_agg_kernel(xs, row3, col3, ew3)
    xs = _bmid(aggp, xs, dinv, b1, W2)
    aggp = _edge_agg_kernel(xs, row3, col3, ew3)
    xs = _bmid(aggp, xs, dinv, b2, W3)
    aggp = _edge_agg_kernel(xs, row3, col3, ew3)
    return _bfin(aggp, xs, dinv, b3)
